# baseline (speedup 1.0000x reference)
import functools

import jax
import jax.numpy as jnp
from jax import lax
from jax.experimental import pallas as pl
from jax.experimental.pallas import tpu as pltpu
from jax.experimental.pallas import tpu_sc as plsc

N_ROWS = 320000
D = 128
NSEG = 1024
NC = 2
NS = 16
NW = NC * NS
G = 16
CHUNK = 256
NGRP = CHUNK // G
NCHUNK = N_ROWS // CHUNK
NPAIR = NCHUNK // 2
NLANE = D // 16
ROWS_PER_TILE_OUT = NSEG // NS


def _sc_body(x_hbm, b_hbm, h_hbm, t_hbm, z_hbm, out_hbm, xb0, xb1, ib0, ib1,
             hb0, hb1, tb0, tb1, sr0, sr1, i16, acc, sem0, sem1, ssem0, ssem1):
    xbufs = (xb0, xb1)
    ibufs = (ib0, ib1)
    hbufs = (hb0, hb1)
    tbufs = (tb0, tb1)
    srows = (sr0, sr1)
    sems = (sem0, sem1)
    ssems = (ssem0, ssem1)

    c = lax.axis_index("c")
    s = lax.axis_index("s")
    w = c * NS + s
    npairs = jnp.where(w < NPAIR % NW, NPAIR // NW + 1, NPAIR // NW)

    def issue(chunk, b):
        base = chunk * CHUNK
        pltpu.make_async_copy(x_hbm.at[pl.ds(base, CHUNK)], xbufs[b], sems[b]).start()
        pltpu.make_async_copy(b_hbm.at[pl.ds(base, CHUNK)], ibufs[b], sems[b]).start()
        pltpu.make_async_copy(
            h_hbm.at[pl.ds(chunk * NGRP, NGRP)], hbufs[b], sems[b]
        ).start()
        pltpu.make_async_copy(
            t_hbm.at[pl.ds(chunk * NGRP, NGRP)], tbufs[b], sems[b]
        ).start()

    def wait_loads(b):
        pltpu.make_async_copy(x_hbm.at[pl.ds(0, CHUNK)], xbufs[b], sems[b]).wait()
        pltpu.make_async_copy(b_hbm.at[pl.ds(0, CHUNK)], ibufs[b], sems[b]).wait()
        pltpu.make_async_copy(h_hbm.at[pl.ds(0, NGRP)], hbufs[b], sems[b]).wait()
        pltpu.make_async_copy(t_hbm.at[pl.ds(0, NGRP)], tbufs[b], sems[b]).wait()

    def drain_scatter(b):
        pltpu.make_async_copy(srows[b], acc.at[hbufs[b]], ssems[b]).wait()

    def process(b):
        zero = jnp.zeros((16,), jnp.float32)
        uni = []
        for g in range(NGRP):
            row0 = g * G
            iv = ibufs[b][pl.ds(row0, G)]
            uni.append(iv[0] == iv[G - 1])
            uniform = uni[g]
            for k in range(NLANE):
                vs = [xbufs[b][row0 + r, pl.ds(k * 16, 16)] for r in range(G)]
                while len(vs) > 1:
                    vs = [vs[i] + vs[i + 1] for i in range(0, len(vs), 2)]
                srows[b][g, pl.ds(k * 16, 16)] = jnp.where(uniform, vs[0], zero)

        for g in range(NGRP):
            @pl.when(jnp.logical_not(uni[g]))
            def _():
                i16[...] = ibufs[b][pl.ds(g * G, G)]
                pltpu.sync_copy(
                    xbufs[b].at[pl.ds(g * G, G)], acc.at[i16], add=True
                )

        pltpu.async_copy(srows[b], acc.at[hbufs[b]], ssems[b], add=True)

    issue(2 * w, 0)
    issue(2 * w + 1, 1)
    pltpu.sync_copy(z_hbm, acc.at[pl.ds(s * ROWS_PER_TILE_OUT, ROWS_PER_TILE_OUT)])
    plsc.subcore_barrier()

    def body(u, carry):
        p = w + NW * u
        for b in range(2):
            wait_loads(b)

            @pl.when(u > 0)
            def _():
                drain_scatter(b)

            process(b)

            @pl.when(p + NW < NPAIR)
            def _():
                issue(2 * (p + NW) + b, b)

        return carry

    lax.fori_loop(0, npairs, body, 0)
    drain_scatter(0)
    drain_scatter(1)

    plsc.subcore_barrier()
    row0 = s * ROWS_PER_TILE_OUT
    pltpu.sync_copy(
        acc.at[pl.ds(row0, ROWS_PER_TILE_OUT)],
        out_hbm.at[pl.ds(c * NSEG + row0, ROWS_PER_TILE_OUT)],
    )


def _combine_body(p_ref, o_ref):
    o_ref[...] = p_ref[0] + p_ref[1]


def kernel(x, batch):
    batch = batch.astype(jnp.int32)
    heads = batch[::G]
    tails = batch[G - 1::G]
    zeros = jnp.zeros((ROWS_PER_TILE_OUT, D), jnp.float32)

    mesh = plsc.VectorSubcoreMesh(core_axis_name="c", subcore_axis_name="s")
    scratch = [
        pltpu.VMEM((CHUNK, D), jnp.float32),
        pltpu.VMEM((CHUNK, D), jnp.float32),
        pltpu.VMEM((CHUNK,), jnp.int32),
        pltpu.VMEM((CHUNK,), jnp.int32),
        pltpu.VMEM((NGRP,), jnp.int32),
        pltpu.VMEM((NGRP,), jnp.int32),
        pltpu.VMEM((NGRP,), jnp.int32),
        pltpu.VMEM((NGRP,), jnp.int32),
        pltpu.VMEM((NGRP, D), jnp.float32),
        pltpu.VMEM((NGRP, D), jnp.float32),
        pltpu.VMEM((G,), jnp.int32),
        pltpu.VMEM_SHARED((NSEG, D), jnp.float32),
        pltpu.SemaphoreType.DMA,
        pltpu.SemaphoreType.DMA,
        pltpu.SemaphoreType.DMA,
        pltpu.SemaphoreType.DMA,
    ]
    partials = pl.kernel(
        _sc_body,
        out_type=jax.ShapeDtypeStruct((NC * NSEG, D), jnp.float32),
        mesh=mesh,
        scratch_types=scratch,
    )(x, batch, heads, tails, zeros)

    out = pl.pallas_call(
        _combine_body,
        out_shape=jax.ShapeDtypeStruct((NSEG, D), jnp.float32),
    )(partials.reshape(NC, NSEG, D))
    return out


# speedup vs baseline: 2.2982x; 2.2982x over previous
import functools

import jax
import jax.numpy as jnp
from jax import lax
from jax.experimental import pallas as pl
from jax.experimental.pallas import tpu as pltpu
from jax.experimental.pallas import tpu_sc as plsc

N_ROWS = 320000
D = 128
NSEG = 1024
NC = 2
NS = 16
NW = NC * NS
ROWS_PER_W = N_ROWS // NW
CHUNK = 80
SUB = 80
NSUB = CHUNK // SUB
NCHUNK = ROWS_PER_W // CHUNK
NBUF = 6
ROWS_PER_TILE_OUT = NSEG // NS


def _sc_body(x_hbm, b_hbm, z_hbm, out_hbm, *refs):
    xbufs = refs[0:NBUF]
    ibufs = refs[NBUF:2 * NBUF]
    n = 2 * NBUF
    isml = tuple(
        tuple(refs[n + b * NSUB + j] for j in range(NSUB)) for b in range(NBUF)
    )
    n += NBUF * NSUB
    acc = refs[n]
    sems = refs[n + 1:n + 1 + NBUF]
    ssems = refs[n + 1 + NBUF:]

    c = lax.axis_index("c")
    s = lax.axis_index("s")
    wid = c * NS + s
    base_w = wid * ROWS_PER_W

    def issue(i, b):
        base = base_w + i * CHUNK
        pltpu.make_async_copy(x_hbm.at[pl.ds(base, CHUNK)], xbufs[b], sems[b]).start()
        pltpu.make_async_copy(b_hbm.at[pl.ds(base, CHUNK)], ibufs[b], sems[b]).start()

    def fire_scatters(b):
        pltpu.make_async_copy(x_hbm.at[pl.ds(base_w, CHUNK)], xbufs[b], sems[b]).wait()
        pltpu.make_async_copy(b_hbm.at[pl.ds(base_w, CHUNK)], ibufs[b], sems[b]).wait()
        for j in range(NSUB):
            for k in range(SUB // 16):
                isml[b][j][pl.ds(k * 16, 16)] = ibufs[b][pl.ds(j * SUB + k * 16, 16)]
        for j in range(NSUB):
            pltpu.async_copy(
                xbufs[b].at[pl.ds(j * SUB, SUB)], acc.at[isml[b][j]], ssems[b],
                add=True,
            )

    def drain_scatters(b):
        for j in range(NSUB):
            pltpu.make_async_copy(
                xbufs[b].at[pl.ds(j * SUB, SUB)], acc.at[isml[b][j]], ssems[b]
            ).wait()

    for b in range(NBUF):
        issue(b, b)
    pltpu.sync_copy(z_hbm, acc.at[pl.ds(s * ROWS_PER_TILE_OUT, ROWS_PER_TILE_OUT)])
    plsc.subcore_barrier()

    def outer(g, carry):
        for b in range(NBUF):
            i = g * NBUF + b
            fire_scatters(b)

            @pl.when(i + NBUF < NCHUNK)
            def _():
                drain_scatters(b)
                issue(i + NBUF, b)

        return carry

    lax.fori_loop(0, NCHUNK // NBUF, outer, 0)
    for r in range(NCHUNK % NBUF):
        fire_scatters(r)
    for b in range(NBUF):
        drain_scatters(b)

    plsc.subcore_barrier()
    row0 = s * ROWS_PER_TILE_OUT
    pltpu.sync_copy(
        acc.at[pl.ds(row0, ROWS_PER_TILE_OUT)],
        out_hbm.at[pl.ds(c * NSEG + row0, ROWS_PER_TILE_OUT)],
    )


def _combine_body(p_ref, o_ref):
    o_ref[...] = p_ref[0] + p_ref[1]


def kernel(x, batch):
    batch = batch.astype(jnp.int32)
    zeros = jnp.zeros((ROWS_PER_TILE_OUT, D), jnp.float32)

    mesh = plsc.VectorSubcoreMesh(core_axis_name="c", subcore_axis_name="s")
    scratch = (
        [pltpu.VMEM((CHUNK, D), jnp.float32) for _ in range(NBUF)]
        + [pltpu.VMEM((CHUNK,), jnp.int32) for _ in range(NBUF)]
        + [pltpu.VMEM((SUB,), jnp.int32) for _ in range(NBUF * NSUB)]
        + [pltpu.VMEM_SHARED((NSEG, D), jnp.float32)]
        + [pltpu.SemaphoreType.DMA for _ in range(2 * NBUF)]
    )
    partials = pl.kernel(
        _sc_body,
        out_type=jax.ShapeDtypeStruct((NC * NSEG, D), jnp.float32),
        mesh=mesh,
        scratch_types=scratch,
    )(x, batch, zeros)

    out = pl.pallas_call(
        _combine_body,
        out_shape=jax.ShapeDtypeStruct((NSEG, D), jnp.float32),
    )(partials.reshape(NC, NSEG, D))
    return out
